# Initial kernel scaffold; baseline (speedup 1.0000x reference)
import jax
import jax.numpy as jnp
from jax.experimental import pallas as pl


def kernel(x, row, col, nnz_weight, bias):
    raise NotImplementedError("write your pallas kernel here")


# speedup vs baseline: 9.8713x; 9.8713x over previous
import functools

import jax
import jax.numpy as jnp
from jax import lax
from jax.experimental import pallas as pl
from jax.experimental.pallas import tpu as pltpu
from jax.experimental.pallas import tpu_sc as plsc

INDIM = 16384
OUTDIM = 16384
NNZ = 262144
BATCH = 64

NC = 2
NS = 16
L = 16

HB = BATCH // NC
K = 1024
EPT = NNZ // NS
NCHUNK = EPT // K
ROWS_PT = OUTDIM // NS
IDXW = 128


@functools.cache
def _build_sc_expander():
    return functools.partial(
        pl.kernel,
        out_type=jax.ShapeDtypeStruct((NC, OUTDIM, HB), jnp.float32),
        mesh=plsc.VectorSubcoreMesh(
            core_axis_name="c", subcore_axis_name="s",
            num_cores=NC, num_subcores=NS,
        ),
        compiler_params=pltpu.CompilerParams(
            use_tc_tiling_on_sc=False, needs_layout_passes=False
        ),
        scratch_types=[
            pltpu.VMEM_SHARED((OUTDIM, HB), jnp.float32),
            pltpu.VMEM((K // IDXW, IDXW), jnp.int32),
            pltpu.VMEM((K // IDXW, IDXW), jnp.int32),
            pltpu.VMEM((K, HB), jnp.float32),
            pltpu.VMEM((K,), jnp.float32),
            pltpu.SemaphoreType.DMA,
            pltpu.SemaphoreType.DMA,
        ],
    )(_sc_expander_body)


def _sc_expander_body(xflat, colm, rowm, w, bias, out, acc_sh, col_v, row_v,
                      vals_v, wv_ref, gsem, ssem):
    cid = lax.axis_index("c")
    sid = lax.axis_index("s")

    r0 = pl.multiple_of(sid * ROWS_PT, ROWS_PT)
    for rb in range(ROWS_PT // K):
        pltpu.sync_copy(bias.at[pl.ds(r0 + rb * K, K)], wv_ref)

        @plsc.parallel_loop(0, K, unroll=8)
        def _(o):
            bv = plsc.load_gather(wv_ref, [jnp.full((L,), o, jnp.int32)])
            vals_v[o, pl.ds(0, L)] = bv
            vals_v[o, pl.ds(L, L)] = bv

        pltpu.sync_copy(vals_v, acc_sh.at[pl.ds(r0 + rb * K, K)])
    plsc.subcore_barrier()

    ebase = pl.multiple_of(sid * EPT, EPT)
    cshift = cid * INDIM

    @pl.loop(0, NCHUNK)
    def _(ck):
        e0 = pl.multiple_of(ebase + ck * K, K)
        m0 = pl.multiple_of(e0 // IDXW, K // IDXW)
        pltpu.sync_copy(colm.at[pl.ds(m0, K // IDXW)], col_v)
        pltpu.sync_copy(rowm.at[pl.ds(m0, K // IDXW)], row_v)
        pltpu.sync_copy(w.at[pl.ds(e0, K)], wv_ref)

        for j in range(K // IDXW):
            for kk in range(IDXW // L):
                sl = pl.ds(kk * L, L)
                col_v[j, sl] = col_v[j, sl] + cshift

        gds = [
            pltpu.async_copy(
                xflat.at[col_v.at[j]],
                vals_v.at[pl.ds(j * IDXW, IDXW)],
                gsem,
            )
            for j in range(K // IDXW)
        ]
        for d in gds:
            d.wait()

        @plsc.parallel_loop(0, K, unroll=8)
        def _(e):
            wv = plsc.load_gather(wv_ref, [jnp.full((L,), e, jnp.int32)])
            vals_v[e, pl.ds(0, L)] = vals_v[e, pl.ds(0, L)] * wv
            vals_v[e, pl.ds(L, L)] = vals_v[e, pl.ds(L, L)] * wv

        sds = [
            pltpu.async_copy(
                vals_v.at[pl.ds(j * IDXW, IDXW)],
                acc_sh.at[row_v.at[j]],
                ssem,
                add=True,
            )
            for j in range(K // IDXW)
        ]
        for d in sds:
            d.wait()

    plsc.subcore_barrier()
    for rb in range(ROWS_PT // K):
        rr = pl.ds(r0 + rb * K, K)
        pltpu.sync_copy(acc_sh.at[rr], out.at[cid, rr])


def kernel(x, row, col, nnz_weight, bias):
    xT = x.T
    xflat = jnp.concatenate([xT[:, :HB], xT[:, HB:]], axis=0)
    colm = col.reshape(NNZ // IDXW, IDXW)
    rowm = row.reshape(NNZ // IDXW, IDXW)
    out2 = _build_sc_expander()(xflat, colm, rowm, nnz_weight, bias)
    y = jnp.concatenate([out2[0], out2[1]], axis=1)
    return y.T


# speedup vs baseline: 14.2737x; 1.4460x over previous
import functools

import jax
import jax.numpy as jnp
from jax import lax
from jax.experimental import pallas as pl
from jax.experimental.pallas import tpu as pltpu
from jax.experimental.pallas import tpu_sc as plsc

INDIM = 16384
OUTDIM = 16384
NNZ = 262144
BATCH = 64

NC = 2
NS = 16
L = 16

HB = BATCH // NC
K = 512
EPT = NNZ // NS
NCHUNK = EPT // K
ROWS_PT = OUTDIM // NS
IDXW = 128
NSUB = K // IDXW
GBYTES = IDXW * HB * 4


@functools.cache
def _build_sc_expander():
    return functools.partial(
        pl.kernel,
        out_type=jax.ShapeDtypeStruct((NC, OUTDIM, HB), jnp.float32),
        mesh=plsc.VectorSubcoreMesh(
            core_axis_name="c", subcore_axis_name="s",
            num_cores=NC, num_subcores=NS,
        ),
        compiler_params=pltpu.CompilerParams(
            use_tc_tiling_on_sc=False, needs_layout_passes=False
        ),
        scratch_types=[
            pltpu.VMEM_SHARED((OUTDIM, HB), jnp.float32),
            pltpu.VMEM((EPT // IDXW, IDXW), jnp.int32),
            pltpu.VMEM((EPT // IDXW, IDXW), jnp.int32),
            pltpu.VMEM((EPT,), jnp.float32),
            pltpu.VMEM((K, HB), jnp.float32),
            pltpu.VMEM((K, HB), jnp.float32),
            pltpu.VMEM((ROWS_PT,), jnp.float32),
            pltpu.SemaphoreType.DMA,
            pltpu.SemaphoreType.DMA,
            pltpu.SemaphoreType.DMA,
            pltpu.SemaphoreType.DMA,
            pltpu.SemaphoreType.DMA,
        ],
    )(_sc_expander_body)


def _sc_expander_body(xflat, colm, rowm, w, bias, out, acc_sh, colt, rowt,
                      wt, vals0, vals1, bias_v, isem, gsem0, gsem1, ssem0,
                      ssem1):
    cid = lax.axis_index("c")
    sid = lax.axis_index("s")
    vals = (vals0, vals1)
    gsem = (gsem0, gsem1)
    ssem = (ssem0, ssem1)

    mrow0 = pl.multiple_of(sid * (EPT // IDXW), EPT // IDXW)
    e0 = pl.multiple_of(sid * EPT, EPT)
    pltpu.async_copy(colm.at[pl.ds(mrow0, EPT // IDXW)], colt, isem)
    pltpu.async_copy(rowm.at[pl.ds(mrow0, EPT // IDXW)], rowt, isem)
    pltpu.async_copy(w.at[pl.ds(e0, EPT)], wt, isem)

    r0 = pl.multiple_of(sid * ROWS_PT, ROWS_PT)
    pltpu.sync_copy(bias.at[pl.ds(r0, ROWS_PT)], bias_v)
    for rb in range(ROWS_PT // K):

        @plsc.parallel_loop(0, K, unroll=8)
        def _(o):
            bv = plsc.load_gather(
                bias_v, [jnp.full((L,), rb * K + o, jnp.int32)]
            )
            vals0[o, pl.ds(0, L)] = bv
            vals0[o, pl.ds(L, L)] = bv

        pltpu.sync_copy(vals0, acc_sh.at[pl.ds(r0 + rb * K, K)])

    pltpu.make_async_copy(colm.at[pl.ds(mrow0, EPT // IDXW)], colt, isem).wait()
    pltpu.make_async_copy(rowm.at[pl.ds(mrow0, EPT // IDXW)], rowt, isem).wait()
    pltpu.make_async_copy(w.at[pl.ds(e0, EPT)], wt, isem).wait()

    cshift = cid * INDIM

    @plsc.parallel_loop(0, EPT // IDXW, unroll=2)
    def _(j):
        for kk in range(IDXW // L):
            sl = pl.ds(kk * L, L)
            colt[j, sl] = colt[j, sl] + cshift

    plsc.subcore_barrier()

    def issue_gathers(ck, b):
        for j in range(NSUB):
            pltpu.async_copy(
                xflat.at[colt.at[ck * NSUB + j]],
                vals[b].at[pl.ds(j * IDXW, IDXW)],
                gsem[b],
            )

    def wait_gathers(ck, b):
        for j in range(NSUB):
            pltpu.make_async_copy(
                xflat.at[colt.at[ck * NSUB + j]],
                vals[b].at[pl.ds(j * IDXW, IDXW)],
                gsem[b],
            ).wait()

    def issue_scatters(ck, b):
        for j in range(NSUB):
            pltpu.async_copy(
                vals[b].at[pl.ds(j * IDXW, IDXW)],
                acc_sh.at[rowt.at[ck * NSUB + j]],
                ssem[b],
                add=True,
            )

    def wait_scatters(ck, b):
        for j in range(NSUB):
            pltpu.make_async_copy(
                vals[b].at[pl.ds(j * IDXW, IDXW)],
                acc_sh.at[rowt.at[ck * NSUB + j]],
                ssem[b],
            ).wait()

    def scale(ck, b):
        vb = vals[b]
        ew0 = ck * K

        @plsc.parallel_loop(0, K, unroll=8)
        def _(e):
            wv = plsc.load_gather(wt, [jnp.full((L,), ew0 + e, jnp.int32)])
            vb[e, pl.ds(0, L)] = vb[e, pl.ds(0, L)] * wv
            vb[e, pl.ds(L, L)] = vb[e, pl.ds(L, L)] * wv

    for ck in range(NCHUNK):
        b = ck & 1
        if ck == 0:
            issue_gathers(0, 0)
        if ck + 1 < NCHUNK:
            if ck >= 1:
                wait_scatters(ck - 1, 1 - b)
            issue_gathers(ck + 1, 1 - b)
        wait_gathers(ck, b)
        scale(ck, b)
        issue_scatters(ck, b)
    wait_scatters(NCHUNK - 2, NCHUNK & 1)
    wait_scatters(NCHUNK - 1, (NCHUNK - 1) & 1)

    plsc.subcore_barrier()
    pltpu.sync_copy(acc_sh.at[pl.ds(r0, ROWS_PT)],
                    out.at[cid, pl.ds(r0, ROWS_PT)])


def kernel(x, row, col, nnz_weight, bias):
    xflat = x.reshape(NC, HB, INDIM).transpose(0, 2, 1).reshape(NC * INDIM, HB)
    colm = col.reshape(NNZ // IDXW, IDXW)
    rowm = row.reshape(NNZ // IDXW, IDXW)
    out2 = _build_sc_expander()(xflat, colm, rowm, nnz_weight, bias)
    return out2.transpose(0, 2, 1).reshape(BATCH, OUTDIM)
